# Initial kernel scaffold; baseline (speedup 1.0000x reference)
import functools

import jax
import jax.numpy as jnp
from jax import lax
from jax.experimental import pallas as pl
from jax.experimental.pallas import tpu as pltpu, tpu_sc as plsc

N = 10000
E = 320000
D = 128

NC = 2
NS = 16
K = 100
NCHUNK = 100
ROWS_PER_TILE = N // NS
CNT_W = 16


def _zero_vmem(ref, nrows, ncols):
  z = jnp.zeros((16,), jnp.float32)
  def body(i, _):
    for jj in range(ncols // 16):
      ref[i, pl.ds(jj * 16, 16)] = z
    return 0
  lax.fori_loop(0, nrows, body, 0)


def _sc_aggregate_body(with_cnt, *refs):
  if with_cnt:
    (y_hbm, src_hbm, dst_hbm, z_hbm, cnt_hbm,
     acc, cntacc, src_idx, dst_idx, rows, zbuf, zcnt, ones,
     sem0, sem1) = refs
  else:
    (y_hbm, src_hbm, dst_hbm, z_hbm,
     acc, src_idx, dst_idx, rows, zbuf,
     sem0, sem1) = refs

  c = lax.axis_index("c")
  s = lax.axis_index("s")

  _zero_vmem(zbuf, 125, 128)
  for t in range(ROWS_PER_TILE // 125):
    pltpu.sync_copy(zbuf, acc.at[pl.ds(s * ROWS_PER_TILE + t * 125, 125), :])
  if with_cnt:
    _zero_vmem(zcnt, ROWS_PER_TILE, CNT_W)
    pltpu.sync_copy(zcnt, cntacc.at[pl.ds(s * ROWS_PER_TILE, ROWS_PER_TILE), :])
    one = jnp.ones((16,), jnp.float32)
    def fill_ones(i, _):
      ones[i, pl.ds(0, 16)] = one
      return 0
    lax.fori_loop(0, K, fill_ones, 0)
  plsc.subcore_barrier()

  pltpu.sync_copy(src_hbm.at[c, s], src_idx)
  pltpu.sync_copy(dst_hbm.at[c, s], dst_idx)

  pltpu.async_copy(y_hbm.at[src_idx.at[0]], rows.at[0], sem0)

  def body(j, _):
    e = 2 * j
    o = e + 1
    pltpu.async_copy(y_hbm.at[src_idx.at[o]], rows.at[1], sem1)
    pltpu.make_async_copy(y_hbm.at[src_idx.at[e]], rows.at[0], sem0).wait()
    pltpu.sync_copy(rows.at[0], acc.at[dst_idx.at[e]], add=True)
    if with_cnt:
      pltpu.sync_copy(ones, cntacc.at[dst_idx.at[e]], add=True)

    @pl.when(o + 1 < NCHUNK)
    def _():
      pltpu.async_copy(y_hbm.at[src_idx.at[o + 1]], rows.at[0], sem0)

    pltpu.make_async_copy(y_hbm.at[src_idx.at[o]], rows.at[1], sem1).wait()
    pltpu.sync_copy(rows.at[1], acc.at[dst_idx.at[o]], add=True)
    if with_cnt:
      pltpu.sync_copy(ones, cntacc.at[dst_idx.at[o]], add=True)
    return 0

  lax.fori_loop(0, NCHUNK // 2, body, 0)
  plsc.subcore_barrier()

  base = s * ROWS_PER_TILE
  pltpu.sync_copy(acc.at[pl.ds(base, ROWS_PER_TILE), :],
                  z_hbm.at[c, pl.ds(base, ROWS_PER_TILE), :])
  if with_cnt:
    pltpu.sync_copy(cntacc.at[pl.ds(base, ROWS_PER_TILE), :],
                    cnt_hbm.at[c, pl.ds(base, ROWS_PER_TILE), :])


def _make_sc_aggregate(with_cnt):
  out_type = [jax.ShapeDtypeStruct((NC, N, D), jnp.float32)]
  scratch = [
      pltpu.VMEM_SHARED((N, D), jnp.float32),
  ]
  if with_cnt:
    out_type.append(jax.ShapeDtypeStruct((NC, N, CNT_W), jnp.float32))
    scratch.append(pltpu.VMEM_SHARED((N, CNT_W), jnp.float32))
  scratch += [
      pltpu.VMEM((NCHUNK, K), jnp.int32),
      pltpu.VMEM((NCHUNK, K), jnp.int32),
      pltpu.VMEM((2, K, D), jnp.float32),
      pltpu.VMEM((125, D), jnp.float32),
  ]
  if with_cnt:
    scratch += [
        pltpu.VMEM((ROWS_PER_TILE, CNT_W), jnp.float32),
        pltpu.VMEM((K, CNT_W), jnp.float32),
    ]
  scratch += [pltpu.SemaphoreType.DMA, pltpu.SemaphoreType.DMA]

  mesh = plsc.VectorSubcoreMesh(core_axis_name="c", subcore_axis_name="s",
                                num_cores=NC, num_subcores=NS)
  return pl.kernel(
      functools.partial(_sc_aggregate_body, with_cnt),
      out_type=out_type,
      mesh=mesh,
      scratch_types=scratch,
  )


_sc_agg_cnt = _make_sc_aggregate(True)
_sc_agg = _make_sc_aggregate(False)


_BT = 1000


def _tc_in_body(x_ref, wl_ref, wr_ref, b_ref, yl_ref, yr_ref):
  x = x_ref[...]
  yl_ref[...] = jnp.dot(x, wl_ref[...], preferred_element_type=jnp.float32)
  yr_ref[...] = (jnp.dot(x, wr_ref[...], preferred_element_type=jnp.float32)
                 + b_ref[...])


def _combine(z_ref, cnt_ref, yr_ref, a_ref):
  zblk = z_ref[...]
  cblk = cnt_ref[...]
  cnt = cblk[0, :, 0:1] + cblk[1, :, 0:1]
  agg = (zblk[0] + zblk[1]) / jnp.maximum(cnt, 1.0)
  h = agg + yr_ref[...]
  return jnp.where(h >= 0.0, h, a_ref[...] * h)


def _tc_mid_body(z_ref, cnt_ref, yr_ref, a_ref, wl_ref, wr_ref, b_ref,
                 yl_out, yr_out):
  h = _combine(z_ref, cnt_ref, yr_ref, a_ref)
  yl_out[...] = jnp.dot(h, wl_ref[...], preferred_element_type=jnp.float32)
  yr_out[...] = (jnp.dot(h, wr_ref[...], preferred_element_type=jnp.float32)
                 + b_ref[...])


def _tc_out_body(z_ref, cnt_ref, yr_ref, a_ref, h_out):
  h_out[...] = _combine(z_ref, cnt_ref, yr_ref, a_ref)


_row_spec = pl.BlockSpec((_BT, D), lambda i: (i, 0))
_w_spec = pl.BlockSpec((D, D), lambda i: (0, 0))
_v_spec = pl.BlockSpec((1, D), lambda i: (0, 0))
_z_spec = pl.BlockSpec((NC, _BT, D), lambda i: (0, i, 0))
_c_spec = pl.BlockSpec((NC, _BT, CNT_W), lambda i: (0, i, 0))

_tc_in = pl.pallas_call(
    _tc_in_body,
    grid=(N // _BT,),
    in_specs=[_row_spec, _w_spec, _w_spec, _v_spec],
    out_specs=[_row_spec, _row_spec],
    out_shape=[jax.ShapeDtypeStruct((N, D), jnp.float32)] * 2,
)

_tc_mid = pl.pallas_call(
    _tc_mid_body,
    grid=(N // _BT,),
    in_specs=[_z_spec, _c_spec, _row_spec, _v_spec, _w_spec, _w_spec, _v_spec],
    out_specs=[_row_spec, _row_spec],
    out_shape=[jax.ShapeDtypeStruct((N, D), jnp.float32)] * 2,
)

_tc_out = pl.pallas_call(
    _tc_out_body,
    grid=(N // _BT,),
    in_specs=[_z_spec, _c_spec, _row_spec, _v_spec],
    out_specs=_row_spec,
    out_shape=jax.ShapeDtypeStruct((N, D), jnp.float32),
)


def kernel(x, edge_index, W1_l, W1_r, b1, a1, W2_l, W2_r, b2, a2,
           W3_l, W3_r, b3, a3):
  src = edge_index[0].astype(jnp.int32).reshape(NC, NS, NCHUNK, K)
  dst = edge_index[1].astype(jnp.int32).reshape(NC, NS, NCHUNK, K)
  b1r = b1.reshape(1, D)
  b2r = b2.reshape(1, D)
  b3r = b3.reshape(1, D)
  a1r = a1.reshape(1, D)
  a2r = a2.reshape(1, D)
  a3r = a3.reshape(1, D)

  y1l, y1r = _tc_in(x, W1_l, W1_r, b1r)
  z1, cnt = _sc_agg_cnt(y1l, src, dst)
  y2l, y2r = _tc_mid(z1, cnt, y1r, a1r, W2_l, W2_r, b2r)
  z2 = _sc_agg(y2l, src, dst)
  y3l, y3r = _tc_mid(z2, cnt, y2r, a2r, W3_l, W3_r, b3r)
  z3 = _sc_agg(y3l, src, dst)
  return _tc_out(z3, cnt, y3r, a3r)


# speedup vs baseline: 10.5728x; 10.5728x over previous
import jax
import jax.numpy as jnp
from jax import lax
from jax.experimental import pallas as pl
from jax.experimental.pallas import tpu as pltpu, tpu_sc as plsc

N = 10000
NPAD = 10240
E = 320000
D = 128

NC = 2
NS = 16
K = 100
NCHUNK = 100
NG = 5
GCH = NCHUNK // NG
ROWS_PER_TILE = NPAD // NS
CNT_W = 16


def _fill_vmem(ref, nrows, ncols, val):
  v = jnp.full((16,), val, jnp.float32)
  def body(i, _):
    for jj in range(ncols // 16):
      ref[i, pl.ds(jj * 16, 16)] = v
    return 0
  lax.fori_loop(0, nrows, body, 0)


def _sc_agg_body(y_hbm, src_hbm, dst_hbm, z_hbm,
                 acc, src_idx, dst_idx, rows, sem0, sem1):
  c = lax.axis_index("c")
  s = lax.axis_index("s")
  base = s * ROWS_PER_TILE

  _fill_vmem(rows.at[0], K, D, 0.0)
  for t in range(ROWS_PER_TILE // K):
    pltpu.sync_copy(rows.at[0], acc.at[pl.ds(base + t * K, K), :])
  pltpu.sync_copy(rows.at[0, pl.ds(0, ROWS_PER_TILE % K)],
                  acc.at[pl.ds(base + (ROWS_PER_TILE // K) * K,
                               ROWS_PER_TILE % K), :])
  plsc.subcore_barrier()

  pltpu.sync_copy(src_hbm.at[c, s], src_idx)

  def group(g, _):
    pltpu.sync_copy(dst_hbm.at[c, s, g], dst_idx)
    cb = g * GCH
    pltpu.async_copy(y_hbm.at[src_idx.at[cb]], rows.at[0], sem0)

    def body(j, _):
      e = cb + 2 * j
      o = e + 1
      pltpu.async_copy(y_hbm.at[src_idx.at[o]], rows.at[1], sem1)
      pltpu.make_async_copy(y_hbm.at[src_idx.at[e]], rows.at[0], sem0).wait()
      pltpu.sync_copy(rows.at[0], acc.at[dst_idx.at[2 * j]], add=True)

      @pl.when(2 * j + 2 < GCH)
      def _():
        pltpu.async_copy(y_hbm.at[src_idx.at[o + 1]], rows.at[0], sem0)

      pltpu.make_async_copy(y_hbm.at[src_idx.at[o]], rows.at[1], sem1).wait()
      pltpu.sync_copy(rows.at[1], acc.at[dst_idx.at[2 * j + 1]], add=True)
      return 0

    lax.fori_loop(0, GCH // 2, body, 0)
    return 0

  lax.fori_loop(0, NG, group, 0)
  plsc.subcore_barrier()

  pltpu.sync_copy(acc.at[pl.ds(base, ROWS_PER_TILE), :],
                  z_hbm.at[c, pl.ds(base, ROWS_PER_TILE), :])


def _sc_cnt_body(dst_hbm, cnt_hbm, cntacc, dst_idx, ones, sem0):
  del sem0
  c = lax.axis_index("c")
  s = lax.axis_index("s")
  base = s * ROWS_PER_TILE

  _fill_vmem(ones, K, D, 0.0)
  for t in range(ROWS_PER_TILE // K):
    pltpu.sync_copy(ones, cntacc.at[pl.ds(base + t * K, K), :])
  pltpu.sync_copy(ones.at[pl.ds(0, ROWS_PER_TILE % K)],
                  cntacc.at[pl.ds(base + (ROWS_PER_TILE // K) * K,
                                  ROWS_PER_TILE % K), :])
  _fill_vmem(ones, K, D, 1.0)
  plsc.subcore_barrier()

  def group(g, _):
    pltpu.sync_copy(dst_hbm.at[c, s, g], dst_idx)

    def body(i, _):
      pltpu.sync_copy(ones, cntacc.at[dst_idx.at[i]], add=True)
      return 0

    lax.fori_loop(0, GCH, body, 0)
    return 0

  lax.fori_loop(0, NG, group, 0)
  plsc.subcore_barrier()

  pltpu.sync_copy(cntacc.at[pl.ds(base, ROWS_PER_TILE), :],
                  cnt_hbm.at[c, pl.ds(base, ROWS_PER_TILE), :])


_sc_mesh = plsc.VectorSubcoreMesh(core_axis_name="c", subcore_axis_name="s",
                                  num_cores=NC, num_subcores=NS)

_sc_agg = pl.kernel(
    _sc_agg_body,
    out_type=jax.ShapeDtypeStruct((NC, NPAD, D), jnp.float32),
    mesh=_sc_mesh,
    scratch_types=[
        pltpu.VMEM_SHARED((NPAD, D), jnp.float32),
        pltpu.VMEM((NCHUNK, K), jnp.int32),
        pltpu.VMEM((GCH, K), jnp.int32),
        pltpu.VMEM((2, K, D), jnp.float32),
        pltpu.SemaphoreType.DMA,
        pltpu.SemaphoreType.DMA,
    ],
)

_sc_cnt = pl.kernel(
    _sc_cnt_body,
    out_type=jax.ShapeDtypeStruct((NC, NPAD, D), jnp.float32),
    mesh=_sc_mesh,
    scratch_types=[
        pltpu.VMEM_SHARED((NPAD, D), jnp.float32),
        pltpu.VMEM((GCH, K), jnp.int32),
        pltpu.VMEM((K, D), jnp.float32),
        pltpu.SemaphoreType.DMA,
    ],
)


_BT = 1000


def _tc_in_body(x_ref, wl_ref, wr_ref, b_ref, yl_ref, yr_ref):
  x = x_ref[...]
  yl_ref[...] = jnp.dot(x, wl_ref[...], preferred_element_type=jnp.float32)
  yr_ref[...] = (jnp.dot(x, wr_ref[...], preferred_element_type=jnp.float32)
                 + b_ref[...])


def _combine(z_ref, cnt_ref, yr_ref, a_ref):
  zblk = z_ref[...]
  cblk = cnt_ref[...]
  cnt = cblk[0, :, 0:1] + cblk[1, :, 0:1]
  agg = (zblk[0] + zblk[1]) / jnp.maximum(cnt, 1.0)
  h = agg + yr_ref[...]
  return jnp.where(h >= 0.0, h, a_ref[...] * h)


def _tc_mid_body(z_ref, cnt_ref, yr_ref, a_ref, wl_ref, wr_ref, b_ref,
                 yl_out, yr_out):
  h = _combine(z_ref, cnt_ref, yr_ref, a_ref)
  yl_out[...] = jnp.dot(h, wl_ref[...], preferred_element_type=jnp.float32)
  yr_out[...] = (jnp.dot(h, wr_ref[...], preferred_element_type=jnp.float32)
                 + b_ref[...])


def _tc_out_body(z_ref, cnt_ref, yr_ref, a_ref, h_out):
  h_out[...] = _combine(z_ref, cnt_ref, yr_ref, a_ref)


_row_spec = pl.BlockSpec((_BT, D), lambda i: (i, 0))
_w_spec = pl.BlockSpec((D, D), lambda i: (0, 0))
_v_spec = pl.BlockSpec((1, D), lambda i: (0, 0))
_z_spec = pl.BlockSpec((NC, _BT, D), lambda i: (0, i, 0))
_c_spec = pl.BlockSpec((NC, _BT, CNT_W), lambda i: (0, i, 0))

_tc_in = pl.pallas_call(
    _tc_in_body,
    grid=(N // _BT,),
    in_specs=[_row_spec, _w_spec, _w_spec, _v_spec],
    out_specs=[_row_spec, _row_spec],
    out_shape=[jax.ShapeDtypeStruct((N, D), jnp.float32)] * 2,
)

_tc_mid = pl.pallas_call(
    _tc_mid_body,
    grid=(N // _BT,),
    in_specs=[_z_spec, _c_spec, _row_spec, _v_spec, _w_spec, _w_spec, _v_spec],
    out_specs=[_row_spec, _row_spec],
    out_shape=[jax.ShapeDtypeStruct((N, D), jnp.float32)] * 2,
)

_tc_out = pl.pallas_call(
    _tc_out_body,
    grid=(N // _BT,),
    in_specs=[_z_spec, _c_spec, _row_spec, _v_spec],
    out_specs=_row_spec,
    out_shape=jax.ShapeDtypeStruct((N, D), jnp.float32),
)


def kernel(x, edge_index, W1_l, W1_r, b1, a1, W2_l, W2_r, b2, a2,
           W3_l, W3_r, b3, a3):
  src = edge_index[0].astype(jnp.int32).reshape(NC, NS, NCHUNK, K)
  dst = edge_index[1].astype(jnp.int32).reshape(NC, NS, NG, GCH, K)
  b1r = b1.reshape(1, D)
  b2r = b2.reshape(1, D)
  b3r = b3.reshape(1, D)
  a1r = a1.reshape(1, D)
  a2r = a2.reshape(1, D)
  a3r = a3.reshape(1, D)

  cnt = _sc_cnt(dst)[:, :, :CNT_W]
  y1l, y1r = _tc_in(x, W1_l, W1_r, b1r)
  z1 = _sc_agg(y1l, src, dst)
  y2l, y2r = _tc_mid(z1, cnt, y1r, a1r, W2_l, W2_r, b2r)
  z2 = _sc_agg(y2l, src, dst)
  y3l, y3r = _tc_mid(z2, cnt, y2r, a2r, W3_l, W3_r, b3r)
  z3 = _sc_agg(y3l, src, dst)
  return _tc_out(z3, cnt, y3r, a3r)
